# Initial kernel scaffold; baseline (speedup 1.0000x reference)
import functools

import jax
import jax.numpy as jnp
from jax import lax
from jax.experimental import pallas as pl
from jax.experimental.pallas import tpu as pltpu
from jax.experimental.pallas import tpu_sc as plsc

N = 10000
E = 320000
F = 128
NPAD = 10240
EPAD = 327680
EROWS = EPAD // 128
DEG_ROWS_PER_TILE = EROWS // 16
TX_ROWS_PER_TILE = EROWS // 32

_MESH = plsc.VectorSubcoreMesh(core_axis_name="c", subcore_axis_name="s")


def _rsqrt16(d):
    i = plsc.bitcast(d, jnp.int32)
    y = plsc.bitcast(jnp.int32(0x5F3759DF) - (i >> 1), jnp.float32)
    half = d * 0.5
    for _ in range(4):
        y = y * (1.5 - half * y * y)
    return jnp.where(d > 0.0, y, 0.0)


def _sc_body(src_hbm, dst_hbm, ew_hbm, x_hbm, out_hbm,
             r2d, c2d, w2d, dis, gbuf, deg_sh, tx1_sh, sem):
    s = lax.axis_index("s")
    c = lax.axis_index("c")

    def _zrow(i, _):
        def _zlane(k, _):
            gbuf[i, pl.ds(k * 16, 16)] = jnp.zeros((16,), jnp.float32)
            return 0
        return lax.fori_loop(0, 8, _zlane, 0)
    lax.fori_loop(0, 128, _zrow, 0)
    for k in range(5):
        pltpu.sync_copy(gbuf, tx1_sh.at[pl.ds(s * 640 + k * 128, 128)])
        pltpu.sync_copy(gbuf.at[0], deg_sh.at[pl.ds(s * 640 + k * 128, 128)])
    plsc.subcore_barrier()

    pltpu.sync_copy(src_hbm.at[pl.ds(s * DEG_ROWS_PER_TILE, DEG_ROWS_PER_TILE)], r2d)
    pltpu.sync_copy(dst_hbm.at[pl.ds(s * DEG_ROWS_PER_TILE, DEG_ROWS_PER_TILE)], c2d)
    pltpu.sync_copy(ew_hbm.at[pl.ds(s * DEG_ROWS_PER_TILE, DEG_ROWS_PER_TILE)], w2d)

    def _deg_row(j, _):
        def _mask(k, _):
            sl = pl.ds(k * 16, 16)
            r = r2d[j, sl]
            cc = c2d[j, sl]
            w = w2d[j, sl]
            w2d[j, sl] = jnp.where(r == cc, 0.0, w)
            return 0
        lax.fori_loop(0, 8, _mask, 0)
        pltpu.sync_copy(w2d.at[j], deg_sh.at[r2d.at[j]], add=True)
        return 0
    lax.fori_loop(0, DEG_ROWS_PER_TILE, _deg_row, 0)
    plsc.subcore_barrier()

    pltpu.sync_copy(deg_sh, dis)

    def _dis(i, _):
        sl = pl.ds(i * 16, 16)
        dis[sl] = _rsqrt16(dis[sl])
        return 0
    lax.fori_loop(0, NPAD // 16, _dis, 0)

    wid = c * 16 + s
    base = wid * TX_ROWS_PER_TILE
    pltpu.sync_copy(src_hbm.at[pl.ds(base, TX_ROWS_PER_TILE)],
                    r2d.at[pl.ds(0, TX_ROWS_PER_TILE)])
    pltpu.sync_copy(dst_hbm.at[pl.ds(base, TX_ROWS_PER_TILE)],
                    c2d.at[pl.ds(0, TX_ROWS_PER_TILE)])
    pltpu.sync_copy(ew_hbm.at[pl.ds(base, TX_ROWS_PER_TILE)],
                    w2d.at[pl.ds(0, TX_ROWS_PER_TILE)])

    def _norm_row(j, _):
        def _norm_lane(k, _):
            sl = pl.ds(k * 16, 16)
            r = r2d[j, sl]
            cc = c2d[j, sl]
            w = w2d[j, sl]
            dr = plsc.load_gather(dis, [r])
            dc = plsc.load_gather(dis, [cc])
            w2d[j, sl] = jnp.where(r == cc, 0.0, -(dr * w * dc))
            return 0
        return lax.fori_loop(0, 8, _norm_lane, 0)
    lax.fori_loop(0, TX_ROWS_PER_TILE, _norm_row, 0)

    def _edge_row(j, _):
        pltpu.async_copy(x_hbm.at[r2d.at[j]], gbuf, sem).wait()

        def _scale(i, _):
            nb = plsc.load_gather(
                w2d, [jnp.full((16,), j, jnp.int32), jnp.full((16,), i, jnp.int32)])

            def _lane(k, _):
                sl = pl.ds(k * 16, 16)
                gbuf[i, sl] = gbuf[i, sl] * nb
                return 0
            return lax.fori_loop(0, 8, _lane, 0)
        lax.fori_loop(0, 128, _scale, 0)
        pltpu.sync_copy(gbuf, tx1_sh.at[c2d.at[j]], add=True)
        return 0
    lax.fori_loop(0, TX_ROWS_PER_TILE, _edge_row, 0)
    plsc.subcore_barrier()

    for k in range(5):
        sl = pl.ds(s * 640 + k * 128, 128)
        pltpu.sync_copy(tx1_sh.at[sl], out_hbm.at[c, sl])


_sc_tx1 = functools.partial(
    pl.kernel,
    out_type=jax.ShapeDtypeStruct((2, NPAD, F), jnp.float32),
    mesh=_MESH,
    scratch_types=[
        pltpu.VMEM((DEG_ROWS_PER_TILE, 128), jnp.int32),
        pltpu.VMEM((DEG_ROWS_PER_TILE, 128), jnp.int32),
        pltpu.VMEM((DEG_ROWS_PER_TILE, 128), jnp.float32),
        pltpu.VMEM((NPAD,), jnp.float32),
        pltpu.VMEM((128, F), jnp.float32),
        pltpu.VMEM_SHARED((NPAD,), jnp.float32),
        pltpu.VMEM_SHARED((NPAD, F), jnp.float32),
        pltpu.SemaphoreType.DMA,
    ],
)(_sc_body)


def _tc_body(x_ref, p0_ref, p1_ref, wx_ref, wt_ref, bz_ref, bh_ref,
             wl_ref, bl_ref, o_ref):
    tx = p0_ref[...] + p1_ref[...]
    h = (jnp.dot(x_ref[...], wx_ref[...], preferred_element_type=jnp.float32)
         + jnp.dot(tx, wt_ref[...], preferred_element_type=jnp.float32))
    z = jax.nn.sigmoid(h[:, :F] + bz_ref[...])
    ht = jnp.tanh(h[:, F:] + bh_ref[...])
    g = jnp.maximum((1.0 - z) * ht, 0.0)
    o_ref[...] = jnp.dot(g, wl_ref[...], preferred_element_type=jnp.float32) + bl_ref[...]


_TC_BLK = 400
_tc_dense = pl.pallas_call(
    _tc_body,
    grid=(N // _TC_BLK,),
    in_specs=[
        pl.BlockSpec((_TC_BLK, F), lambda i: (i, 0)),
        pl.BlockSpec((_TC_BLK, F), lambda i: (i, 0)),
        pl.BlockSpec((_TC_BLK, F), lambda i: (i, 0)),
        pl.BlockSpec((F, 2 * F), lambda i: (0, 0)),
        pl.BlockSpec((F, 2 * F), lambda i: (0, 0)),
        pl.BlockSpec((1, F), lambda i: (0, 0)),
        pl.BlockSpec((1, F), lambda i: (0, 0)),
        pl.BlockSpec((F, 1), lambda i: (0, 0)),
        pl.BlockSpec((1, 1), lambda i: (0, 0)),
    ],
    out_specs=pl.BlockSpec((_TC_BLK, 1), lambda i: (i, 0)),
    out_shape=jax.ShapeDtypeStruct((N, 1), jnp.float32),
)


def kernel(x, edge_index, edge_weight,
           W0_xz, W1_xz, b_xz, W0_hz, W1_hz, b_hz,
           W0_xr, W1_xr, b_xr, W0_hr, W1_hr, b_hr,
           W0_xh, W1_xh, b_xh, W0_hh, W1_hh, b_hh,
           Wl, bl):
    pad = EPAD - E
    src = jnp.concatenate([edge_index[0], jnp.zeros((pad,), jnp.int32)]).reshape(EROWS, 128)
    dst = jnp.concatenate([edge_index[1], jnp.zeros((pad,), jnp.int32)]).reshape(EROWS, 128)
    ew = jnp.concatenate([edge_weight, jnp.zeros((pad,), jnp.float32)]).reshape(EROWS, 128)

    partials = _sc_tx1(src, dst, ew, x)

    wx = jnp.concatenate([W0_xz, W0_xh], axis=1)
    wt = jnp.concatenate([W1_xz, W1_xh], axis=1)
    bz = (b_xz + b_hz).reshape(1, F)
    bh = (b_xh + b_hh).reshape(1, F)

    return _tc_dense(x, partials[0, :N], partials[1, :N], wx, wt, bz, bh,
                     Wl, bl.reshape(1, 1))


# speedup vs baseline: 14.5051x; 14.5051x over previous
import functools

import jax
import jax.numpy as jnp
from jax import lax
from jax.experimental import pallas as pl
from jax.experimental.pallas import tpu as pltpu
from jax.experimental.pallas import tpu_sc as plsc

N = 10000
E = 320000
F = 128
FH = F // 2
NPAD = 10240
EPAD = 327680
EROWS = EPAD // 128
RPT = EROWS // 16

_MESH = plsc.VectorSubcoreMesh(core_axis_name="c", subcore_axis_name="s")


def _rsqrt16(d):
    i = lax.bitcast_convert_type(d, jnp.int32)
    y = lax.bitcast_convert_type(jnp.int32(0x5F3759DF) - (i >> 1), jnp.float32)
    half = d * 0.5
    for _ in range(4):
        y = y * (1.5 - half * y * y)
    return jnp.where(d > 0.0, y, 0.0)


def _sc_body(src_hbm, dst_hbm, ew_hbm, xl_hbm, xr_hbm, out_hbm,
             r2d, c2d, w2d, dis, hbuf, deg_sh, tx1_sh, sem):
    s = lax.axis_index("s")
    c = lax.axis_index("c")

    def _zrow(i, _):
        for k in range(FH // 16):
            hbuf[i, pl.ds(k * 16, 16)] = jnp.zeros((16,), jnp.float32)
        return 0
    lax.fori_loop(0, 128, _zrow, 0)
    for k in range(5):
        pltpu.sync_copy(hbuf, tx1_sh.at[pl.ds(s * 640 + k * 128, 128)])
    for k in range(10):
        pltpu.sync_copy(hbuf.at[0], deg_sh.at[pl.ds(s * 640 + k * 64, 64)])
    plsc.subcore_barrier()

    pltpu.sync_copy(src_hbm.at[pl.ds(s * RPT, RPT)], r2d)
    pltpu.sync_copy(dst_hbm.at[pl.ds(s * RPT, RPT)], c2d)
    pltpu.sync_copy(ew_hbm.at[pl.ds(s * RPT, RPT)], w2d)

    def _deg_row(j, _):
        def _mask(k, _):
            sl = pl.ds(k * 16, 16)
            w2d[j, sl] = jnp.where(r2d[j, sl] == c2d[j, sl], 0.0, w2d[j, sl])
            return 0
        lax.fori_loop(0, 8, _mask, 0)
        pltpu.sync_copy(w2d.at[j], deg_sh.at[r2d.at[j]], add=True)
        return 0
    lax.fori_loop(0, RPT, _deg_row, 0)
    plsc.subcore_barrier()

    pltpu.sync_copy(deg_sh, dis)

    def _dis(i, _):
        sl = pl.ds(i * 16, 16)
        dis[sl] = _rsqrt16(dis[sl])
        return 0
    lax.fori_loop(0, NPAD // 16, _dis, 0)

    def _norm_row(j, _):
        def _norm_lane(k, _):
            sl = pl.ds(k * 16, 16)
            dr = plsc.load_gather(dis, [r2d[j, sl]])
            dc = plsc.load_gather(dis, [c2d[j, sl]])
            w2d[j, sl] = -(dr * w2d[j, sl] * dc)
            return 0
        return lax.fori_loop(0, 8, _norm_lane, 0)
    lax.fori_loop(0, RPT, _norm_row, 0)

    def _edge_rows(x_hbm):
        def _edge_row(j, _):
            pltpu.async_copy(x_hbm.at[r2d.at[j]], hbuf, sem).wait()

            def _scale(i, _):
                nb = plsc.load_gather(
                    w2d, [jnp.full((16,), j, jnp.int32),
                          jnp.full((16,), i, jnp.int32)])
                for k in range(FH // 16):
                    sl = pl.ds(k * 16, 16)
                    hbuf[i, sl] = hbuf[i, sl] * nb
                return 0
            lax.fori_loop(0, 128, _scale, 0)
            pltpu.sync_copy(hbuf, tx1_sh.at[c2d.at[j]], add=True)
            return 0
        lax.fori_loop(0, RPT, _edge_row, 0)

    @pl.when(c == 0)
    def _():
        _edge_rows(xl_hbm)

    @pl.when(c == 1)
    def _():
        _edge_rows(xr_hbm)

    plsc.subcore_barrier()

    for k in range(5):
        sl = pl.ds(s * 640 + k * 128, 128)
        pltpu.sync_copy(tx1_sh.at[sl], out_hbm.at[c, sl])


_sc_tx1 = functools.partial(
    pl.kernel,
    out_type=jax.ShapeDtypeStruct((2, NPAD, FH), jnp.float32),
    mesh=_MESH,
    compiler_params=pltpu.CompilerParams(
        needs_layout_passes=False, use_tc_tiling_on_sc=False),
    scratch_types=[
        pltpu.VMEM((RPT, 128), jnp.int32),
        pltpu.VMEM((RPT, 128), jnp.int32),
        pltpu.VMEM((RPT, 128), jnp.float32),
        pltpu.VMEM((NPAD,), jnp.float32),
        pltpu.VMEM((128, FH), jnp.float32),
        pltpu.VMEM_SHARED((NPAD,), jnp.float32),
        pltpu.VMEM_SHARED((NPAD, FH), jnp.float32),
        pltpu.SemaphoreType.DMA,
    ],
)(_sc_body)


def _tc_body(x_ref, p0_ref, p1_ref, wx_ref, wt_ref, bz_ref, bh_ref,
             wl_ref, bl_ref, o_ref):
    tx = jnp.concatenate([p0_ref[...], p1_ref[...]], axis=1)
    h = (jnp.dot(x_ref[...], wx_ref[...], preferred_element_type=jnp.float32)
         + jnp.dot(tx, wt_ref[...], preferred_element_type=jnp.float32))
    z = jax.nn.sigmoid(h[:, :F] + bz_ref[...])
    ht = jnp.tanh(h[:, F:] + bh_ref[...])
    g = jnp.maximum((1.0 - z) * ht, 0.0)
    o_ref[...] = jnp.dot(g, wl_ref[...], preferred_element_type=jnp.float32) + bl_ref[...]


_TC_BLK = 400
_tc_dense = pl.pallas_call(
    _tc_body,
    grid=(N // _TC_BLK,),
    in_specs=[
        pl.BlockSpec((_TC_BLK, F), lambda i: (i, 0)),
        pl.BlockSpec((_TC_BLK, FH), lambda i: (i, 0)),
        pl.BlockSpec((_TC_BLK, FH), lambda i: (i, 0)),
        pl.BlockSpec((F, 2 * F), lambda i: (0, 0)),
        pl.BlockSpec((F, 2 * F), lambda i: (0, 0)),
        pl.BlockSpec((1, F), lambda i: (0, 0)),
        pl.BlockSpec((1, F), lambda i: (0, 0)),
        pl.BlockSpec((F, 1), lambda i: (0, 0)),
        pl.BlockSpec((1, 1), lambda i: (0, 0)),
    ],
    out_specs=pl.BlockSpec((_TC_BLK, 1), lambda i: (i, 0)),
    out_shape=jax.ShapeDtypeStruct((N, 1), jnp.float32),
)


def kernel(x, edge_index, edge_weight,
           W0_xz, W1_xz, b_xz, W0_hz, W1_hz, b_hz,
           W0_xr, W1_xr, b_xr, W0_hr, W1_hr, b_hr,
           W0_xh, W1_xh, b_xh, W0_hh, W1_hh, b_hh,
           Wl, bl):
    pad = EPAD - E
    src = jnp.concatenate([edge_index[0], jnp.zeros((pad,), jnp.int32)]).reshape(EROWS, 128)
    dst = jnp.concatenate([edge_index[1], jnp.zeros((pad,), jnp.int32)]).reshape(EROWS, 128)
    ew = jnp.concatenate([edge_weight, jnp.zeros((pad,), jnp.float32)]).reshape(EROWS, 128)
    xl = x[:, :FH]
    xr = x[:, FH:]
    halves = _sc_tx1(src, dst, ew, xl, xr)

    wx = jnp.concatenate([W0_xz, W0_xh], axis=1)
    wt = jnp.concatenate([W1_xz, W1_xh], axis=1)
    bz = (b_xz + b_hz).reshape(1, F)
    bh = (b_xh + b_hh).reshape(1, F)

    return _tc_dense(x, halves[0, :N], halves[1, :N], wx, wt, bz, bh,
                     Wl, bl.reshape(1, 1))
